# Initial kernel scaffold; baseline (speedup 1.0000x reference)
import functools

import jax
import jax.numpy as jnp
from jax import lax
from jax.experimental import pallas as pl
from jax.experimental.pallas import tpu as pltpu
from jax.experimental.pallas import tpu_sc as plsc

_N = 10000
_E = 320000
_D = 128
_H = 64
_A = 0.5

_NC = 2
_NS = 16
_NW = _NC * _NS
_EPW = _E // _NW
_CH = 80
_NCHUNK = _EPW // _CH
_RPT = _N // _NS

_F32 = jnp.float32


def _mesh():
    return plsc.VectorSubcoreMesh(
        core_axis_name="c", subcore_axis_name="s",
        num_cores=_NC, num_subcores=_NS)


def _degcnt_pass(dst, ew, zeros16):
    @functools.partial(
        pl.kernel,
        out_type=jax.ShapeDtypeStruct((_NC, _N, 16), _F32),
        mesh=_mesh(),
        scratch_types=[
            pltpu.VMEM((_CH,), jnp.int32),
            pltpu.VMEM((_CH,), _F32),
            pltpu.VMEM((_CH, 16), _F32),
            pltpu.VMEM_SHARED((_N, 16), _F32),
            pltpu.SemaphoreType.DMA,
        ])
    def k(dst_hbm, ew_hbm, z_hbm, out_hbm, dst_v, w_v, rows_v, acc, sem):
        c = lax.axis_index("c")
        s = lax.axis_index("s")
        wid = s * _NC + c
        r0 = s * _RPT
        pltpu.sync_copy(z_hbm.at[pl.ds(r0, _RPT)], acc.at[pl.ds(r0, _RPT)])

        def zrow(e, carry):
            rows_v[e, :] = jnp.zeros((16,), _F32)
            return carry
        lax.fori_loop(0, _CH, zrow, 0)
        plsc.subcore_barrier()

        ebase = wid * _EPW
        iota = lax.iota(jnp.int32, 16)
        col0 = jnp.zeros((16,), jnp.int32)
        col1 = jnp.ones((16,), jnp.int32)
        ones = jnp.ones((16,), _F32)

        def chunk(i, carry):
            off = ebase + i * _CH
            pltpu.sync_copy(dst_hbm.at[pl.ds(off, _CH)], dst_v)
            pltpu.sync_copy(ew_hbm.at[pl.ds(off, _CH)], w_v)
            for g in range(_CH // 16):
                wv = w_v[pl.ds(g * 16, 16)]
                rix = iota + (g * 16)
                plsc.store_scatter(rows_v, [rix, col0], wv)
                plsc.store_scatter(rows_v, [rix, col1], ones)
            pltpu.sync_copy(rows_v, acc.at[dst_v], add=True)
            return carry
        lax.fori_loop(0, _NCHUNK, chunk, 0)

        plsc.subcore_barrier()
        pltpu.sync_copy(acc.at[pl.ds(r0, _RPT)], out_hbm.at[c, pl.ds(r0, _RPT)])

    return k(dst, ew, zeros16)


def _edge_pass(width, scaled, table, src, dst, ew, zeros):
    @functools.partial(
        pl.kernel,
        out_type=jax.ShapeDtypeStruct((_NC, _N, width), _F32),
        mesh=_mesh(),
        scratch_types=[
            pltpu.VMEM((_CH,), jnp.int32),
            pltpu.VMEM((_CH,), jnp.int32),
            pltpu.VMEM((_CH,), _F32),
            pltpu.VMEM((_CH, width), _F32),
            pltpu.VMEM_SHARED((_N, width), _F32),
            pltpu.SemaphoreType.DMA,
        ])
    def k(table_hbm, src_hbm, dst_hbm, ew_hbm, z_hbm, out_hbm,
          src_v, dst_v, w_v, rows_v, acc, sem):
        c = lax.axis_index("c")
        s = lax.axis_index("s")
        wid = s * _NC + c
        r0 = s * _RPT
        pltpu.sync_copy(z_hbm.at[pl.ds(r0, _RPT)], acc.at[pl.ds(r0, _RPT)])
        plsc.subcore_barrier()

        ebase = wid * _EPW

        def chunk(i, carry):
            off = ebase + i * _CH
            pltpu.sync_copy(src_hbm.at[pl.ds(off, _CH)], src_v)
            pltpu.sync_copy(dst_hbm.at[pl.ds(off, _CH)], dst_v)
            if scaled:
                pltpu.sync_copy(ew_hbm.at[pl.ds(off, _CH)], w_v)
            pltpu.async_copy(table_hbm.at[src_v], rows_v, sem).wait()
            if scaled:
                def esc(e, ecarry):
                    wb = plsc.load_gather(w_v, [jnp.full((16,), e, jnp.int32)])
                    for cg in range(width // 16):
                        sl = pl.ds(cg * 16, 16)
                        rows_v[e, sl] = rows_v[e, sl] * wb
                    return ecarry
                lax.fori_loop(0, _CH, esc, 0)
            pltpu.sync_copy(rows_v, acc.at[dst_v], add=True)
            return carry
        lax.fori_loop(0, _NCHUNK, chunk, 0)

        plsc.subcore_barrier()
        pltpu.sync_copy(acc.at[pl.ds(r0, _RPT)], out_hbm.at[c, pl.ds(r0, _RPT)])

    return k(table, src, dst, ew, zeros)


def _dot(a, b):
    return jnp.dot(a, b, preferred_element_type=_F32,
                   precision=lax.Precision.HIGHEST)


def _tc1(acc_a, x, W1):
    def body(acc_ref, x_ref, w1_ref, dis_ref, cnt_ref, h1_ref, h1p_ref):
        a = acc_ref[0] + acc_ref[1]
        deg = a[:, 0:1] + 1.0
        dis = lax.rsqrt(deg)
        h1 = _dot(x_ref[...], w1_ref[...])
        dis_ref[...] = dis
        cnt_ref[...] = a[:, 1:2]
        h1_ref[...] = h1
        h1p_ref[...] = h1 * dis
    return pl.pallas_call(
        body,
        out_shape=(jax.ShapeDtypeStruct((_N, 1), _F32),
                   jax.ShapeDtypeStruct((_N, 1), _F32),
                   jax.ShapeDtypeStruct((_N, _H), _F32),
                   jax.ShapeDtypeStruct((_N, _H), _F32)))(acc_a, x, W1)


def _tc2(acc_b, dis, h1, b1, W2):
    def body(acc_ref, dis_ref, h1_ref, b1_ref, w2_ref, h2_ref, h2p_ref):
        dis = dis_ref[...]
        s1 = acc_ref[0] + acc_ref[1]
        z1 = jnp.maximum(dis * s1 + (dis * dis) * h1_ref[...] + b1_ref[...], 0.0)
        h2 = _dot(z1, w2_ref[...])
        h2_ref[...] = h2
        h2p_ref[...] = h2 * dis
    return pl.pallas_call(
        body,
        out_shape=(jax.ShapeDtypeStruct((_N, _H), _F32),
                   jax.ShapeDtypeStruct((_N, _H), _F32)))(acc_b, dis, h1, b1, W2)


def _tc3(acc_c, dis, h2, b2, x, Wa1, ba1, Wa2, ba2):
    def body(acc_ref, dis_ref, h2_ref, b2_ref, x_ref,
             wa1_ref, ba1_ref, wa2_ref, ba2_ref, z_ref, err_ref):
        dis = dis_ref[...]
        s2 = acc_ref[0] + acc_ref[1]
        z = jnp.maximum(dis * s2 + (dis * dis) * h2_ref[...] + b2_ref[...], 0.0)
        u = jnp.maximum(_dot(z, wa1_ref[...]) + ba1_ref[...], 0.0)
        x_hat = jax.nn.sigmoid(_dot(u, wa2_ref[...]) + ba2_ref[...])
        d = x_hat - x_ref[...]
        z_ref[...] = z
        err_ref[...] = jnp.sqrt(jnp.sum(d * d, axis=1, keepdims=True) + 1e-12)
    return pl.pallas_call(
        body,
        out_shape=(jax.ShapeDtypeStruct((_N, _H), _F32),
                   jax.ShapeDtypeStruct((_N, 1), _F32)))(
            acc_c, dis, h2, b2, x, Wa1, ba1, Wa2, ba2)


def _tc4(acc_d1, acc_d2, cnt, attr_err, Wh1, bh1, Wh2, bh2):
    def body(d1_ref, d2_ref, cnt_ref, attr_ref,
             wh1_ref, bh1_ref, wh2_ref, bh2_ref, score_ref):
        inv = 1.0 / jnp.maximum(cnt_ref[...], 1.0)
        z_bar = (d1_ref[0] + d1_ref[1]) * inv
        m_x = (d2_ref[0] + d2_ref[1]) * inv
        v = jnp.maximum(_dot(z_bar, wh1_ref[...]) + bh1_ref[...], 0.0)
        x_homo = jax.nn.sigmoid(_dot(v, wh2_ref[...]) + bh2_ref[...])
        d = x_homo - m_x
        homo = jnp.sqrt(jnp.sum(d * d, axis=1) + 1e-12)
        score_ref[...] = _A * attr_ref[:, 0] + (1.0 - _A) * homo
    return pl.pallas_call(
        body,
        out_shape=jax.ShapeDtypeStruct((_N,), _F32))(
            acc_d1, acc_d2, cnt, attr_err, Wh1, bh1, Wh2, bh2)


def kernel(x, edge_index, edge_weight, W1, b1, W2, b2,
           Wa1, ba1, Wa2, ba2, Wh1, bh1, Wh2, bh2):
    src = edge_index[0]
    dst = edge_index[1]
    z16 = jnp.zeros((_N, 16), _F32)
    z64 = jnp.zeros((_N, _H), _F32)
    z128 = jnp.zeros((_N, _D), _F32)

    acc_a = _degcnt_pass(dst, edge_weight, z16)
    acc_d2 = _edge_pass(_D, False, x, src, dst, edge_weight, z128)

    dis, cnt, h1, h1p = _tc1(acc_a, x, W1)
    acc_b = _edge_pass(_H, True, h1p, src, dst, edge_weight, z64)
    h2, h2p = _tc2(acc_b, dis, h1, b1, W2)
    acc_c = _edge_pass(_H, True, h2p, src, dst, edge_weight, z64)
    z, attr_err = _tc3(acc_c, dis, h2, b2, x, Wa1, ba1, Wa2, ba2)
    acc_d1 = _edge_pass(_H, False, z, src, dst, edge_weight, z64)
    score = _tc4(acc_d1, acc_d2, cnt, attr_err, Wh1, bh1, Wh2, bh2)
    return score


# speedup vs baseline: 6.7694x; 6.7694x over previous
import functools

import jax
import jax.numpy as jnp
from jax import lax
from jax.experimental import pallas as pl
from jax.experimental.pallas import tpu as pltpu
from jax.experimental.pallas import tpu_sc as plsc

_N = 10000
_E = 320000
_D = 128
_H = 64
_A = 0.5

_NC = 2
_NS = 16
_NW = _NC * _NS
_EPW = _E // _NW
_CH = 80
_NCHUNK = _EPW // _CH
_NP = 10240
_RPT = _NP // _NS

_F32 = jnp.float32


def _mesh():
    return plsc.VectorSubcoreMesh(
        core_axis_name="c", subcore_axis_name="s",
        num_cores=_NC, num_subcores=_NS)


def _degcnt_pass(dst, ew, zeros16):
    @functools.partial(
        pl.kernel,
        out_type=jax.ShapeDtypeStruct((_NC, _NP, 16), _F32),
        mesh=_mesh(),
        compiler_params=pltpu.CompilerParams(use_tc_tiling_on_sc=False),
        scratch_types=[
            pltpu.VMEM((_CH,), jnp.int32),
            pltpu.VMEM((_CH,), _F32),
            pltpu.VMEM((_CH, 16), _F32),
            pltpu.VMEM_SHARED((_NP, 16), _F32),
            pltpu.SemaphoreType.DMA,
        ])
    def k(dst_hbm, ew_hbm, z_hbm, out_hbm, dst_v, w_v, rows_v, acc, sem):
        c = lax.axis_index("c")
        s = lax.axis_index("s")
        wid = s * _NC + c
        r0 = s * _RPT
        pltpu.sync_copy(z_hbm.at[pl.ds(r0, _RPT)], acc.at[pl.ds(r0, _RPT)])

        plsc.subcore_barrier()

        ebase = wid * _EPW
        iota = lax.iota(jnp.int32, 16)
        ones = jnp.ones((16,), _F32)
        zero = jnp.zeros((16,), _F32)

        def chunk(i, carry):
            off = ebase + i * _CH
            pltpu.sync_copy(dst_hbm.at[pl.ds(off, _CH)], dst_v)
            pltpu.sync_copy(ew_hbm.at[pl.ds(off, _CH)], w_v)

            def erow(g, ecarry):
                v16 = w_v[pl.ds(g * 16, 16)]
                for j in range(16):
                    wb = jnp.full((16,), v16[j], _F32)
                    rows_v[g * 16 + j, :] = jnp.where(
                        iota == 0, wb, jnp.where(iota == 1, ones, zero))
                return ecarry
            lax.fori_loop(0, _CH // 16, erow, 0)
            pltpu.sync_copy(rows_v, acc.at[dst_v], add=True)
            return carry
        lax.fori_loop(0, _NCHUNK, chunk, 0)

        plsc.subcore_barrier()
        pltpu.sync_copy(acc.at[pl.ds(r0, _RPT)], out_hbm.at[c, pl.ds(r0, _RPT)])

    return k(dst, ew, zeros16)


def _edge_pass(width, scaled, table, src, dst, ew, zeros):
    @functools.partial(
        pl.kernel,
        out_type=jax.ShapeDtypeStruct((_NC, _NP, width), _F32),
        mesh=_mesh(),
        compiler_params=pltpu.CompilerParams(use_tc_tiling_on_sc=False),
        scratch_types=[
            pltpu.VMEM((_CH,), jnp.int32),
            pltpu.VMEM((_CH,), jnp.int32),
            pltpu.VMEM((_CH,), _F32),
            pltpu.VMEM((_CH, width), _F32),
            pltpu.VMEM_SHARED((_NP, width), _F32),
            pltpu.SemaphoreType.DMA,
        ])
    def k(table_hbm, src_hbm, dst_hbm, ew_hbm, z_hbm, out_hbm,
          src_v, dst_v, w_v, rows_v, acc, sem):
        c = lax.axis_index("c")
        s = lax.axis_index("s")
        wid = s * _NC + c
        r0 = s * _RPT
        pltpu.sync_copy(z_hbm.at[pl.ds(r0, _RPT)], acc.at[pl.ds(r0, _RPT)])
        plsc.subcore_barrier()

        ebase = wid * _EPW

        def chunk(i, carry):
            off = ebase + i * _CH
            pltpu.sync_copy(src_hbm.at[pl.ds(off, _CH)], src_v)
            pltpu.sync_copy(dst_hbm.at[pl.ds(off, _CH)], dst_v)
            if scaled:
                pltpu.sync_copy(ew_hbm.at[pl.ds(off, _CH)], w_v)
            pltpu.async_copy(table_hbm.at[src_v], rows_v, sem).wait()
            if scaled:
                def esc(g, ecarry):
                    v16 = w_v[pl.ds(g * 16, 16)]
                    for j in range(16):
                        wb = jnp.full((16,), v16[j], _F32)
                        e = g * 16 + j
                        for cg in range(width // 16):
                            sl = pl.ds(cg * 16, 16)
                            rows_v[e, sl] = rows_v[e, sl] * wb
                    return ecarry
                lax.fori_loop(0, _CH // 16, esc, 0)
            pltpu.sync_copy(rows_v, acc.at[dst_v], add=True)
            return carry
        lax.fori_loop(0, _NCHUNK, chunk, 0)

        plsc.subcore_barrier()
        pltpu.sync_copy(acc.at[pl.ds(r0, _RPT)], out_hbm.at[c, pl.ds(r0, _RPT)])

    return k(table, src, dst, ew, zeros)


_TC_PARAMS = pltpu.CompilerParams(vmem_limit_bytes=100 * 1024 * 1024)


def _dot(a, b):
    return jnp.dot(a, b, preferred_element_type=_F32,
                   precision=lax.Precision.HIGHEST)


def _tc1(acc_a, x, W1):
    def body(acc_ref, x_ref, w1_ref, dis_ref, cnt_ref, h1_ref, h1p_ref):
        a = acc_ref[0, :_N] + acc_ref[1, :_N]
        deg = a[:, 0:1] + 1.0
        dis = lax.rsqrt(deg)
        h1 = _dot(x_ref[...], w1_ref[...])
        dis_ref[...] = dis
        cnt_ref[...] = a[:, 1:2]
        h1_ref[...] = h1
        h1p_ref[...] = h1 * dis
    return pl.pallas_call(
        body,
        compiler_params=_TC_PARAMS,
        out_shape=(jax.ShapeDtypeStruct((_N, 1), _F32),
                   jax.ShapeDtypeStruct((_N, 1), _F32),
                   jax.ShapeDtypeStruct((_N, _H), _F32),
                   jax.ShapeDtypeStruct((_N, _H), _F32)))(acc_a, x, W1)


def _tc2(acc_b, dis, h1, b1, W2):
    def body(acc_ref, dis_ref, h1_ref, b1_ref, w2_ref, h2_ref, h2p_ref):
        dis = dis_ref[...]
        s1 = acc_ref[0, :_N] + acc_ref[1, :_N]
        z1 = jnp.maximum(dis * s1 + (dis * dis) * h1_ref[...] + b1_ref[...], 0.0)
        h2 = _dot(z1, w2_ref[...])
        h2_ref[...] = h2
        h2p_ref[...] = h2 * dis
    return pl.pallas_call(
        body,
        compiler_params=_TC_PARAMS,
        out_shape=(jax.ShapeDtypeStruct((_N, _H), _F32),
                   jax.ShapeDtypeStruct((_N, _H), _F32)))(acc_b, dis, h1, b1, W2)


def _tc3(acc_c, dis, h2, b2, x, Wa1, ba1, Wa2, ba2):
    def body(acc_ref, dis_ref, h2_ref, b2_ref, x_ref,
             wa1_ref, ba1_ref, wa2_ref, ba2_ref, z_ref, err_ref):
        dis = dis_ref[...]
        s2 = acc_ref[0, :_N] + acc_ref[1, :_N]
        z = jnp.maximum(dis * s2 + (dis * dis) * h2_ref[...] + b2_ref[...], 0.0)
        u = jnp.maximum(_dot(z, wa1_ref[...]) + ba1_ref[...], 0.0)
        x_hat = jax.nn.sigmoid(_dot(u, wa2_ref[...]) + ba2_ref[...])
        d = x_hat - x_ref[...]
        z_ref[...] = z
        err_ref[...] = jnp.sqrt(jnp.sum(d * d, axis=1, keepdims=True) + 1e-12)
    return pl.pallas_call(
        body,
        compiler_params=_TC_PARAMS,
        out_shape=(jax.ShapeDtypeStruct((_N, _H), _F32),
                   jax.ShapeDtypeStruct((_N, 1), _F32)))(
            acc_c, dis, h2, b2, x, Wa1, ba1, Wa2, ba2)


def _tc4(acc_d1, acc_d2, cnt, attr_err, Wh1, bh1, Wh2, bh2):
    def body(d1_ref, d2_ref, cnt_ref, attr_ref,
             wh1_ref, bh1_ref, wh2_ref, bh2_ref, score_ref):
        inv = 1.0 / jnp.maximum(cnt_ref[...], 1.0)
        z_bar = (d1_ref[0, :_N] + d1_ref[1, :_N]) * inv
        m_x = (d2_ref[0, :_N] + d2_ref[1, :_N]) * inv
        v = jnp.maximum(_dot(z_bar, wh1_ref[...]) + bh1_ref[...], 0.0)
        x_homo = jax.nn.sigmoid(_dot(v, wh2_ref[...]) + bh2_ref[...])
        d = x_homo - m_x
        homo = jnp.sqrt(jnp.sum(d * d, axis=1) + 1e-12)
        score_ref[...] = _A * attr_ref[:, 0] + (1.0 - _A) * homo
    return pl.pallas_call(
        body,
        compiler_params=_TC_PARAMS,
        out_shape=jax.ShapeDtypeStruct((_N,), _F32))(
            acc_d1, acc_d2, cnt, attr_err, Wh1, bh1, Wh2, bh2)


def kernel(x, edge_index, edge_weight, W1, b1, W2, b2,
           Wa1, ba1, Wa2, ba2, Wh1, bh1, Wh2, bh2):
    src = edge_index[0]
    dst = edge_index[1]
    z16 = jnp.zeros((_NP, 16), _F32)
    z64 = jnp.zeros((_NP, _H), _F32)
    z128 = jnp.zeros((_NP, _D), _F32)

    acc_a = _degcnt_pass(dst, edge_weight, z16)
    acc_d2 = _edge_pass(_D, False, x, src, dst, edge_weight, z128)

    dis, cnt, h1, h1p = _tc1(acc_a, x, W1)
    acc_b = _edge_pass(_H, True, h1p, src, dst, edge_weight, z64)
    h2, h2p = _tc2(acc_b, dis, h1, b1, W2)
    acc_c = _edge_pass(_H, True, h2p, src, dst, edge_weight, z64)
    z, attr_err = _tc3(acc_c, dis, h2, b2, x, Wa1, ba1, Wa2, ba2)
    acc_d1 = _edge_pass(_H, False, z, src, dst, edge_weight, z64)
    score = _tc4(acc_d1, acc_d2, cnt, attr_err, Wh1, bh1, Wh2, bh2)
    return score


# speedup vs baseline: 9.0336x; 1.3345x over previous
import functools

import jax
import jax.numpy as jnp
from jax import lax
from jax.experimental import pallas as pl
from jax.experimental.pallas import tpu as pltpu
from jax.experimental.pallas import tpu_sc as plsc

_N = 10000
_E = 320000
_D = 128
_H = 64
_A = 0.5

_NC = 2
_NS = 16
_NW = _NC * _NS
_CH = 128
_NCHUNK = 79
_EPW = _CH * _NCHUNK
_EPAD = _NW * _EPW
_NP = 10240
_RPT = _NP // _NS

_F32 = jnp.float32


def _mesh():
    return plsc.VectorSubcoreMesh(
        core_axis_name="c", subcore_axis_name="s",
        num_cores=_NC, num_subcores=_NS)


def _degcnt_pass(dst_r, ew_r, zeros16):
    @functools.partial(
        pl.kernel,
        out_type=jax.ShapeDtypeStruct((_NC, _NP, 16), _F32),
        mesh=_mesh(),
        compiler_params=pltpu.CompilerParams(use_tc_tiling_on_sc=False),
        scratch_types=[
            pltpu.VMEM((_NCHUNK, _CH), jnp.int32),
            pltpu.VMEM((_NCHUNK, _CH), _F32),
            pltpu.VMEM((_CH, 16), _F32),
            pltpu.VMEM_SHARED((_NP, 16), _F32),
            pltpu.SemaphoreType.DMA,
        ])
    def k(dst_hbm, ew_hbm, z_hbm, out_hbm, dst_all, w_all, rows_v, acc, sem):
        c = lax.axis_index("c")
        s = lax.axis_index("s")
        wid = s * _NC + c
        r0 = s * _RPT
        pltpu.sync_copy(z_hbm.at[pl.ds(r0, _RPT)], acc.at[pl.ds(r0, _RPT)])
        pltpu.sync_copy(dst_hbm.at[wid], dst_all)
        pltpu.sync_copy(ew_hbm.at[wid], w_all)
        plsc.subcore_barrier()

        iota = lax.iota(jnp.int32, 16)
        ones = jnp.ones((16,), _F32)
        zero = jnp.zeros((16,), _F32)

        def chunk(i, carry):
            def erow(g, ecarry):
                v16 = w_all[i, pl.ds(g * 16, 16)]
                for j in range(16):
                    wb = jnp.full((16,), v16[j], _F32)
                    rows_v[g * 16 + j, :] = jnp.where(
                        iota == 0, wb, jnp.where(iota == 1, ones, zero))
                return ecarry
            lax.fori_loop(0, _CH // 16, erow, 0)
            pltpu.sync_copy(rows_v, acc.at[dst_all.at[i]], add=True)
            return carry
        lax.fori_loop(0, _NCHUNK, chunk, 0)

        plsc.subcore_barrier()
        pltpu.sync_copy(acc.at[pl.ds(r0, _RPT)], out_hbm.at[c, pl.ds(r0, _RPT)])

    return k(dst_r, ew_r, zeros16)


def _edge_pass(width, scaled, table, src_r, dst_r, ew_r, zeros):
    @functools.partial(
        pl.kernel,
        out_type=jax.ShapeDtypeStruct((_NC, _NP, width), _F32),
        mesh=_mesh(),
        compiler_params=pltpu.CompilerParams(use_tc_tiling_on_sc=False),
        scratch_types=[
            pltpu.VMEM((_NCHUNK, _CH), jnp.int32),
            pltpu.VMEM((_NCHUNK, _CH), jnp.int32),
            pltpu.VMEM((_NCHUNK, _CH), _F32),
            pltpu.VMEM((_CH, width), _F32),
            pltpu.VMEM((_CH, width), _F32),
            pltpu.VMEM_SHARED((_NP, width), _F32),
            pltpu.SemaphoreType.DMA,
            pltpu.SemaphoreType.DMA,
        ])
    def k(table_hbm, src_hbm, dst_hbm, ew_hbm, z_hbm, out_hbm,
          src_all, dst_all, w_all, rows0, rows1, acc, sem0, sem1):
        c = lax.axis_index("c")
        s = lax.axis_index("s")
        wid = s * _NC + c
        r0 = s * _RPT
        pltpu.sync_copy(z_hbm.at[pl.ds(r0, _RPT)], acc.at[pl.ds(r0, _RPT)])
        pltpu.sync_copy(src_hbm.at[wid], src_all)
        pltpu.sync_copy(dst_hbm.at[wid], dst_all)
        if scaled:
            pltpu.sync_copy(ew_hbm.at[wid], w_all)
        plsc.subcore_barrier()

        bufs = (rows0, rows1)
        sems = (sem0, sem1)

        def scale_rows(i, buf):
            def grp(g, gcarry):
                v16 = w_all[i, pl.ds(g * 16, 16)]
                for j in range(16):
                    wb = jnp.full((16,), v16[j], _F32)
                    e = g * 16 + j
                    for cg in range(width // 16):
                        sl = pl.ds(cg * 16, 16)
                        buf[e, sl] = buf[e, sl] * wb
                return gcarry
            lax.fori_loop(0, _CH // 16, grp, 0)

        pltpu.async_copy(table_hbm.at[src_all.at[0]], rows0, sem0)

        def outer(g, carry):
            for b in range(2):
                i = g * 2 + b
                nxt = i + 1

                @pl.when(nxt < _NCHUNK)
                def _():
                    pltpu.async_copy(
                        table_hbm.at[src_all.at[nxt]], bufs[1 - b], sems[1 - b])
                pltpu.make_async_copy(
                    table_hbm.at[src_all.at[i]], bufs[b], sems[b]).wait()
                if scaled:
                    scale_rows(i, bufs[b])
                pltpu.sync_copy(bufs[b], acc.at[dst_all.at[i]], add=True)
            return carry
        lax.fori_loop(0, _NCHUNK // 2, outer, 0)

        t = _NCHUNK - 1
        pltpu.make_async_copy(table_hbm.at[src_all.at[t]], rows0, sem0).wait()
        if scaled:
            scale_rows(t, rows0)
        pltpu.sync_copy(rows0, acc.at[dst_all.at[t]], add=True)

        plsc.subcore_barrier()
        pltpu.sync_copy(acc.at[pl.ds(r0, _RPT)], out_hbm.at[c, pl.ds(r0, _RPT)])

    return k(table, src_r, dst_r, ew_r, zeros)


_TC_PARAMS = pltpu.CompilerParams(vmem_limit_bytes=100 * 1024 * 1024)
_BN = 2000
_GRID = _N // _BN


def _dot(a, b):
    return jnp.dot(a, b, preferred_element_type=_F32,
                   precision=lax.Precision.HIGHEST)


def _bs_acc(width):
    return pl.BlockSpec((_NC, _BN, width), lambda i: (0, i, 0))


def _bs_rows(width):
    return pl.BlockSpec((_BN, width), lambda i: (i, 0))


def _bs_full2(shape):
    return pl.BlockSpec(shape, lambda i: (0, 0))


def _bs_full1(shape):
    return pl.BlockSpec(shape, lambda i: (0,))


def _tc1(acc_a, x, W1):
    def body(acc_ref, x_ref, w1_ref, dis_ref, cnt_ref, h1_ref, h1p_ref):
        a = acc_ref[0] + acc_ref[1]
        deg = a[:, 0:1] + 1.0
        dis = lax.rsqrt(deg)
        h1 = _dot(x_ref[...], w1_ref[...])
        dis_ref[...] = dis
        cnt_ref[...] = a[:, 1:2]
        h1_ref[...] = h1
        h1p_ref[...] = h1 * dis
    return pl.pallas_call(
        body,
        grid=(_GRID,),
        in_specs=[_bs_acc(16), _bs_rows(_D), _bs_full2((_D, _H))],
        out_specs=(_bs_rows(1), _bs_rows(1), _bs_rows(_H), _bs_rows(_H)),
        compiler_params=_TC_PARAMS,
        out_shape=(jax.ShapeDtypeStruct((_N, 1), _F32),
                   jax.ShapeDtypeStruct((_N, 1), _F32),
                   jax.ShapeDtypeStruct((_N, _H), _F32),
                   jax.ShapeDtypeStruct((_N, _H), _F32)))(acc_a, x, W1)


def _tc2(acc_b, dis, h1, b1, W2):
    def body(acc_ref, dis_ref, h1_ref, b1_ref, w2_ref, h2_ref, h2p_ref):
        dis = dis_ref[...]
        s1 = acc_ref[0] + acc_ref[1]
        z1 = jnp.maximum(dis * s1 + (dis * dis) * h1_ref[...] + b1_ref[...], 0.0)
        h2 = _dot(z1, w2_ref[...])
        h2_ref[...] = h2
        h2p_ref[...] = h2 * dis
    return pl.pallas_call(
        body,
        grid=(_GRID,),
        in_specs=[_bs_acc(_H), _bs_rows(1), _bs_rows(_H), _bs_full1((_H,)),
                  _bs_full2((_H, _H))],
        out_specs=(_bs_rows(_H), _bs_rows(_H)),
        compiler_params=_TC_PARAMS,
        out_shape=(jax.ShapeDtypeStruct((_N, _H), _F32),
                   jax.ShapeDtypeStruct((_N, _H), _F32)))(acc_b, dis, h1, b1, W2)


def _tc3(acc_c, dis, h2, b2, x, Wa1, ba1, Wa2, ba2):
    def body(acc_ref, dis_ref, h2_ref, b2_ref, x_ref,
             wa1_ref, ba1_ref, wa2_ref, ba2_ref, z_ref, err_ref):
        dis = dis_ref[...]
        s2 = acc_ref[0] + acc_ref[1]
        z = jnp.maximum(dis * s2 + (dis * dis) * h2_ref[...] + b2_ref[...], 0.0)
        u = jnp.maximum(_dot(z, wa1_ref[...]) + ba1_ref[...], 0.0)
        x_hat = jax.nn.sigmoid(_dot(u, wa2_ref[...]) + ba2_ref[...])
        d = x_hat - x_ref[...]
        z_ref[...] = z
        err_ref[...] = jnp.sqrt(jnp.sum(d * d, axis=1, keepdims=True) + 1e-12)
    return pl.pallas_call(
        body,
        grid=(_GRID,),
        in_specs=[_bs_acc(_H), _bs_rows(1), _bs_rows(_H), _bs_full1((_H,)),
                  _bs_rows(_D), _bs_full2((_H, _H)), _bs_full1((_H,)),
                  _bs_full2((_H, _D)), _bs_full1((_D,))],
        out_specs=(_bs_rows(_H), _bs_rows(1)),
        compiler_params=_TC_PARAMS,
        out_shape=(jax.ShapeDtypeStruct((_N, _H), _F32),
                   jax.ShapeDtypeStruct((_N, 1), _F32)))(
            acc_c, dis, h2, b2, x, Wa1, ba1, Wa2, ba2)


def _tc4(acc_d1, acc_d2a, acc_d2b, cnt, attr_err, Wh1, bh1, Wh2, bh2):
    def body(d1_ref, d2a_ref, d2b_ref, cnt_ref, attr_ref,
             wh1_ref, bh1_ref, wh2_ref, bh2_ref, score_ref):
        inv = 1.0 / jnp.maximum(cnt_ref[...], 1.0)
        z_bar = (d1_ref[0] + d1_ref[1]) * inv
        m_xa = (d2a_ref[0] + d2a_ref[1]) * inv
        m_xb = (d2b_ref[0] + d2b_ref[1]) * inv
        v = jnp.maximum(_dot(z_bar, wh1_ref[...]) + bh1_ref[...], 0.0)
        x_homo = jax.nn.sigmoid(_dot(v, wh2_ref[...]) + bh2_ref[...])
        da = x_homo[:, :_H] - m_xa
        db = x_homo[:, _H:] - m_xb
        homo = jnp.sqrt(jnp.sum(da * da, axis=1) + jnp.sum(db * db, axis=1)
                        + 1e-12)
        score_ref[...] = (_A * attr_ref[:, 0]
                          + (1.0 - _A) * homo)[:, None]
    return pl.pallas_call(
        body,
        grid=(_GRID,),
        in_specs=[_bs_acc(_H), _bs_acc(_H), _bs_acc(_H), _bs_rows(1),
                  _bs_rows(1), _bs_full2((_H, _H)), _bs_full1((_H,)),
                  _bs_full2((_H, _D)), _bs_full1((_D,))],
        out_specs=_bs_rows(1),
        compiler_params=_TC_PARAMS,
        out_shape=jax.ShapeDtypeStruct((_N, 1), _F32))(
            acc_d1, acc_d2a, acc_d2b, cnt, attr_err, Wh1, bh1, Wh2, bh2)


def kernel(x, edge_index, edge_weight, W1, b1, W2, b2,
           Wa1, ba1, Wa2, ba2, Wh1, bh1, Wh2, bh2):
    src = edge_index[0]
    dst = edge_index[1]
    npad = _EPAD - _E
    src_p = jnp.concatenate([src, jnp.zeros((npad,), jnp.int32)])
    dst_p = jnp.concatenate(
        [dst, _N + (jnp.arange(npad, dtype=jnp.int32) % (_NP - _N))])
    ew_p = jnp.concatenate([edge_weight, jnp.zeros((npad,), _F32)])
    src_r = src_p.reshape(_NW, _NCHUNK, _CH)
    dst_r = dst_p.reshape(_NW, _NCHUNK, _CH)
    ew_r = ew_p.reshape(_NW, _NCHUNK, _CH)

    z16 = jnp.zeros((_NP, 16), _F32)
    z64 = jnp.zeros((_NP, _H), _F32)

    acc_a = _degcnt_pass(dst_r, ew_r, z16)
    acc_d2a = _edge_pass(_H, False, x[:, :_H], src_r, dst_r, ew_r, z64)
    acc_d2b = _edge_pass(_H, False, x[:, _H:], src_r, dst_r, ew_r, z64)

    dis, cnt, h1, h1p = _tc1(acc_a, x, W1)
    acc_b = _edge_pass(_H, True, h1p, src_r, dst_r, ew_r, z64)
    h2, h2p = _tc2(acc_b, dis, h1, b1, W2)
    acc_c = _edge_pass(_H, True, h2p, src_r, dst_r, ew_r, z64)
    z, attr_err = _tc3(acc_c, dis, h2, b2, x, Wa1, ba1, Wa2, ba2)
    acc_d1 = _edge_pass(_H, False, z, src_r, dst_r, ew_r, z64)
    score = _tc4(acc_d1, acc_d2a, acc_d2b, cnt, attr_err, Wh1, bh1, Wh2, bh2)
    return score[:, 0]
